# baseline (speedup 1.0000x reference)
import functools

import jax
import jax.numpy as jnp
from jax import lax
from jax.experimental import pallas as pl
from jax.experimental.pallas import tpu as pltpu
from jax.experimental.pallas import tpu_sc as plsc

EMBEDDING_DIM = 64
PACKED_DIM = 2 * EMBEDDING_DIM
NUM_CORES = 2
NUM_SUBCORES = 16
NUM_WORKERS = NUM_CORES * NUM_SUBCORES
NSLOT = 5
CHUNK = 256
VB = 4096


def _row_major_table(wt):
    dim, vocab = wt.shape
    nblk = pl.cdiv(vocab, 2 * VB)
    hb = nblk * VB

    def body(lo_ref, hi_ref, o_ref):
        o_ref[...] = jnp.concatenate([lo_ref[...].T, hi_ref[...].T], axis=1)

    return pl.pallas_call(
        body,
        grid=(nblk,),
        in_specs=[
            pl.BlockSpec((dim, VB), lambda i: (0, i)),
            pl.BlockSpec(
                (dim, VB),
                lambda i, _n=nblk, _l=(vocab - 1) // VB: (
                    0,
                    jnp.minimum(i + _n, _l),
                ),
            ),
        ],
        out_specs=pl.BlockSpec((VB, PACKED_DIM), lambda i: (i, 0)),
        out_shape=jax.ShapeDtypeStruct((hb, PACKED_DIM), jnp.float32),
        compiler_params=pltpu.CompilerParams(dimension_semantics=("parallel",)),
    )(wt, wt)


def _sc_gather(table, idx, batch):
    n = idx.shape[0]
    half = batch // 2
    hv = table.shape[0] // 2
    per_worker = n // NUM_WORKERS
    n_chunks = per_worker // CHUNK
    n_groups = n_chunks // NSLOT
    mesh = plsc.VectorSubcoreMesh(core_axis_name="c", subcore_axis_name="s")

    @functools.partial(
        pl.kernel,
        mesh=mesh,
        compiler_params=pltpu.CompilerParams(use_tc_tiling_on_sc=False),
        out_type=jax.ShapeDtypeStruct((n // 2, 2, EMBEDDING_DIM), jnp.float32),
        scratch_types=[
            pltpu.VMEM((per_worker,), jnp.int32),
        ]
        + [pltpu.VMEM((CHUNK, EMBEDDING_DIM), jnp.float32) for _ in range(NSLOT)]
        + [pltpu.SemaphoreType.DMA for _ in range(2 * NSLOT)],
    )
    def gather_k(table_hbm, idx_hbm, out_hbm, idx_v, *scratch):
        bufs = scratch[:NSLOT]
        gsems = scratch[NSLOT : 2 * NSLOT]
        wsems = scratch[2 * NSLOT :]
        wid = lax.axis_index("s") * NUM_CORES + lax.axis_index("c")
        base = wid * per_worker
        pltpu.sync_copy(idx_hbm.at[pl.ds(base, per_worker)], idx_v)

        @pl.loop(0, per_worker, step=16)
        def _(i):
            v = idx_v[pl.ds(i, 16)]
            idx_v[pl.ds(i, 16)] = 2 * v - jnp.where(
                v >= hv, jnp.int32(2 * hv - 1), jnp.int32(0)
            )

        def start_gather(c, s):
            return pltpu.async_copy(
                table_hbm.at[idx_v.at[pl.ds(c * CHUNK, CHUNK)]], bufs[s], gsems[s]
            )

        def start_write(c, s):
            o = base + c * CHUNK
            sq = o // batch
            rem = o % batch
            p = rem // half
            j0 = rem % half
            t0 = sq * half + j0
            return pltpu.async_copy(
                bufs[s], out_hbm.at[pl.ds(t0, CHUNK), p], wsems[s]
            )

        for s in range(NSLOT):
            start_gather(s, s)

        @pl.loop(0, n_groups)
        def _(g):
            L = g * NSLOT
            handles = []
            for s in range(NSLOT):
                pltpu.make_async_copy(
                    table_hbm.at[idx_v.at[pl.ds(0, CHUNK)]], bufs[s], gsems[s]
                ).wait()
                handles.append(start_write(L + s, s))
            for s in range(NSLOT):
                handles[s].wait()

                @pl.when(g < n_groups - 1)
                def _():
                    start_gather(L + s + NSLOT, s)

    return gather_k(table, idx)


def _to_batch_minor(packed, seq, batch):
    half = batch // 2
    SB = 10

    def body(in_ref, o_ref):
        for i in range(SB):
            v = in_ref[i]
            o_ref[i] = jnp.concatenate(
                [v[:, :EMBEDDING_DIM].T, v[:, EMBEDDING_DIM:].T], axis=1
            )

    return pl.pallas_call(
        body,
        grid=(seq // SB,),
        in_specs=[pl.BlockSpec((SB, half, PACKED_DIM), lambda s: (s, 0, 0))],
        out_specs=pl.BlockSpec((SB, EMBEDDING_DIM, batch), lambda s: (s, 0, 0)),
        out_shape=jax.ShapeDtypeStruct((seq, EMBEDDING_DIM, batch), jnp.float32),
        compiler_params=pltpu.CompilerParams(dimension_semantics=("parallel",)),
    )(packed)


def kernel(x, weight):
    batch, seq = x.shape
    n = batch * seq
    half = batch // 2
    idx = x.T.reshape(n)
    packed_table = _row_major_table(weight.T)
    table = packed_table.reshape(2 * packed_table.shape[0], EMBEDDING_DIM)
    out3 = _sc_gather(table, idx, batch)
    packed = out3.reshape(seq, half, PACKED_DIM)
    p = _to_batch_minor(packed, seq, batch)
    return jnp.transpose(p, (2, 0, 1))


# speedup vs baseline: 1.0117x; 1.0117x over previous
import functools

import jax
import jax.numpy as jnp
from jax import lax
from jax.experimental import pallas as pl
from jax.experimental.pallas import tpu as pltpu
from jax.experimental.pallas import tpu_sc as plsc

EMBEDDING_DIM = 64
PACKED_DIM = 2 * EMBEDDING_DIM
NUM_CORES = 2
NUM_SUBCORES = 16
NUM_WORKERS = NUM_CORES * NUM_SUBCORES
NSLOT = 5
CHUNK = 256
VB = 4096


def _row_major_table(wt):
    dim, vocab = wt.shape
    nblk = pl.cdiv(vocab, 2 * VB)
    hb = nblk * VB

    def body(lo_ref, hi_ref, o_ref):
        o_ref[...] = jnp.concatenate([lo_ref[...].T, hi_ref[...].T], axis=1)

    return pl.pallas_call(
        body,
        grid=(nblk,),
        in_specs=[
            pl.BlockSpec((dim, VB), lambda i: (0, i)),
            pl.BlockSpec(
                (dim, VB),
                lambda i, _n=nblk, _l=(vocab - 1) // VB: (
                    0,
                    jnp.minimum(i + _n, _l),
                ),
            ),
        ],
        out_specs=pl.BlockSpec((VB, PACKED_DIM), lambda i: (i, 0)),
        out_shape=jax.ShapeDtypeStruct((hb, PACKED_DIM), jnp.float32),
        compiler_params=pltpu.CompilerParams(dimension_semantics=("parallel",)),
    )(wt, wt)


def _sc_gather(table, idx, batch):
    n = idx.shape[0]
    half = batch // 2
    hv = table.shape[0] // 2
    per_worker = n // NUM_WORKERS
    n_chunks = per_worker // CHUNK
    n_groups = n_chunks // NSLOT
    mesh = plsc.VectorSubcoreMesh(core_axis_name="c", subcore_axis_name="s")

    @functools.partial(
        pl.kernel,
        mesh=mesh,
        compiler_params=pltpu.CompilerParams(use_tc_tiling_on_sc=False),
        out_type=jax.ShapeDtypeStruct((n // 2, 2, EMBEDDING_DIM), jnp.float32),
        scratch_types=[
            pltpu.VMEM((per_worker,), jnp.int32),
        ]
        + [pltpu.VMEM((CHUNK, EMBEDDING_DIM), jnp.float32) for _ in range(NSLOT)]
        + [pltpu.SemaphoreType.DMA for _ in range(2 * NSLOT)],
    )
    def gather_k(table_hbm, idx_hbm, out_hbm, idx_v, *scratch):
        bufs = scratch[:NSLOT]
        gsems = scratch[NSLOT : 2 * NSLOT]
        wsems = scratch[2 * NSLOT :]
        wid = lax.axis_index("s") * NUM_CORES + lax.axis_index("c")
        base = wid * per_worker
        pltpu.sync_copy(idx_hbm.at[pl.ds(base, per_worker)], idx_v)

        @pl.loop(0, per_worker, step=16)
        def _(i):
            v = idx_v[pl.ds(i, 16)]
            idx_v[pl.ds(i, 16)] = 2 * v - jnp.where(
                v >= hv, jnp.int32(2 * hv - 1), jnp.int32(0)
            )

        def start_gather(c, s):
            return pltpu.async_copy(
                table_hbm.at[idx_v.at[pl.ds(c * CHUNK, CHUNK)]], bufs[s], gsems[s]
            )

        def start_write(c, s):
            o = base + c * CHUNK
            sq = o // batch
            rem = o % batch
            p = rem // half
            j0 = rem % half
            t0 = sq * half + j0
            return pltpu.async_copy(
                bufs[s], out_hbm.at[pl.ds(t0, CHUNK), p], wsems[s]
            )

        for s in range(NSLOT):
            start_gather(s, s)

        @pl.loop(0, n_groups)
        def _(g):
            L = g * NSLOT
            handles = []
            for s in range(NSLOT):
                pltpu.make_async_copy(
                    table_hbm.at[idx_v.at[pl.ds(0, CHUNK)]], bufs[s], gsems[s]
                ).wait()
                handles.append(start_write(L + s, s))
            for s in range(NSLOT):
                handles[s].wait()

                @pl.when(g < n_groups - 1)
                def _():
                    start_gather(L + s + NSLOT, s)

    return gather_k(table, idx)


def _to_batch_minor(packed, seq, batch):
    half = batch // 2
    SB = 5

    def body(in_ref, o_ref):
        for i in range(SB):
            v = in_ref[i]
            o_ref[i] = jnp.concatenate(
                [v[:, :EMBEDDING_DIM].T, v[:, EMBEDDING_DIM:].T], axis=1
            )

    return pl.pallas_call(
        body,
        grid=(seq // SB,),
        in_specs=[pl.BlockSpec((SB, half, PACKED_DIM), lambda s: (s, 0, 0))],
        out_specs=pl.BlockSpec((SB, EMBEDDING_DIM, batch), lambda s: (s, 0, 0)),
        out_shape=jax.ShapeDtypeStruct((seq, EMBEDDING_DIM, batch), jnp.float32),
        compiler_params=pltpu.CompilerParams(dimension_semantics=("parallel",)),
    )(packed)


def kernel(x, weight):
    batch, seq = x.shape
    n = batch * seq
    half = batch // 2
    idx = x.T.reshape(n)
    packed_table = _row_major_table(weight.T)
    table = packed_table.reshape(2 * packed_table.shape[0], EMBEDDING_DIM)
    out3 = _sc_gather(table, idx, batch)
    packed = out3.reshape(seq, half, PACKED_DIM)
    p = _to_batch_minor(packed, seq, batch)
    return jnp.transpose(p, (2, 0, 1))
